# baseline (device time: 177702 ns/iter reference)
import functools

import jax
import jax.numpy as jnp
from jax import lax
from jax.experimental import pallas as pl
from jax.experimental.pallas import tpu as pltpu

N_DEV = 4
SQ = 1024
SKV = 1024
HQ_LOCAL = 8
DH = 128
D_MODEL = 1024
SCALE = 0.08838834764831843


def _body(x_ref, wq_ref, k_ref, v_ref, wo_ref, out_ref,
          ctx_ref, comm_ref, send_sems, recv_sems):
    my = lax.axis_index("i")
    left = lax.rem(my - 1 + N_DEV, N_DEV)
    right = lax.rem(my + 1, N_DEV)

    barrier_sem = pltpu.get_barrier_semaphore()
    for nbr in (left, right):
        pl.semaphore_signal(
            barrier_sem, inc=1,
            device_id=(nbr,), device_id_type=pl.DeviceIdType.MESH,
        )
    pl.semaphore_wait(barrier_sem, 2)

    q = jnp.dot(x_ref[:, :], wq_ref[:, :],
                preferred_element_type=jnp.float32)
    q = q.astype(jnp.bfloat16)

    qi = lax.broadcasted_iota(jnp.int32, (SQ, SKV), 0)
    ki = lax.broadcasted_iota(jnp.int32, (SQ, SKV), 1)
    mask = (jnp.abs(qi - ki) <= 128) | (ki < 32) | (qi < 32)

    for h in range(HQ_LOCAL):
        qh = q[:, h * DH:(h + 1) * DH]
        kh = k_ref[h]
        s = lax.dot_general(
            qh, kh,
            dimension_numbers=(((1,), (1,)), ((), ())),
            preferred_element_type=jnp.float32,
        ) * SCALE
        s = jnp.where(mask, s, -1e9)
        m = jnp.max(s, axis=-1, keepdims=True)
        w = jnp.exp(s - m)
        w = w / jnp.sum(w, axis=-1, keepdims=True)
        ch = jnp.dot(w.astype(jnp.bfloat16), v_ref[h],
                     preferred_element_type=jnp.float32)
        ctx_ref[:, h * DH:(h + 1) * DH] = ch.astype(jnp.bfloat16)

    partial = jnp.dot(ctx_ref[:, :], wo_ref[:, :],
                      preferred_element_type=jnp.float32)
    out_ref[:, :] = partial
    comm_ref[0, :, :] = partial

    for h in range(N_DEV - 1):
        rdma = pltpu.make_async_remote_copy(
            src_ref=comm_ref.at[h],
            dst_ref=comm_ref.at[h + 1],
            send_sem=send_sems.at[h],
            recv_sem=recv_sems.at[h],
            device_id=(right,),
            device_id_type=pl.DeviceIdType.MESH,
        )
        rdma.start()
        rdma.wait()
        out_ref[:, :] += comm_ref[h + 1, :, :]


def kernel(x, Wq, K_ext, V_ext, Wo):
    my = lax.axis_index("i")

    xb = x[0].astype(jnp.bfloat16)
    Wqb = Wq.astype(jnp.bfloat16)
    Wob = Wo.astype(jnp.bfloat16)
    Kh = lax.dynamic_slice_in_dim(K_ext[0], my * HQ_LOCAL, HQ_LOCAL, axis=1)
    Vh = lax.dynamic_slice_in_dim(V_ext[0], my * HQ_LOCAL, HQ_LOCAL, axis=1)
    Kh = jnp.transpose(Kh, (1, 0, 2)).astype(jnp.bfloat16)
    Vh = jnp.transpose(Vh, (1, 0, 2)).astype(jnp.bfloat16)

    out = pl.pallas_call(
        _body,
        out_shape=jax.ShapeDtypeStruct((SQ, D_MODEL), jnp.float32),
        in_specs=[pl.BlockSpec(memory_space=pltpu.VMEM)] * 5,
        out_specs=pl.BlockSpec(memory_space=pltpu.VMEM),
        scratch_shapes=[
            pltpu.VMEM((SQ, HQ_LOCAL * DH), jnp.bfloat16),
            pltpu.VMEM((N_DEV, SQ, D_MODEL), jnp.float32),
            pltpu.SemaphoreType.DMA((N_DEV - 1,)),
            pltpu.SemaphoreType.DMA((N_DEV - 1,)),
        ],
        compiler_params=pltpu.CompilerParams(collective_id=0),
    )(xb, Wqb, Kh, Vh, Wob)
    return out[None]


# device time: 80060 ns/iter; 2.2196x vs baseline; 2.2196x over previous
import jax
import jax.numpy as jnp
from jax import lax
from jax.experimental import pallas as pl
from jax.experimental.pallas import tpu as pltpu

N_DEV = 4
SQ = 1024
SKV = 1024
HQ_LOCAL = 8
DH = 128
D_MODEL = 1024
CHUNK = SQ // N_DEV
SCALE = 0.08838834764831843


def _mod4(v):
    return lax.rem(v + 2 * N_DEV, N_DEV)


def _body(x_ref, wq_ref, k_ref, v_ref, wo_ref, out_ref,
          ctx_ref, pb_ref, fwd_ref, rs_recv_ref, mychunk_ref, ag_recv_ref,
          rs_send_sems, rs_recv_sems, ag_send_sems, ag_recv_sems):
    my = lax.axis_index("i")
    left = _mod4(my - 1)
    right = _mod4(my + 1)

    barrier_sem = pltpu.get_barrier_semaphore()
    for nbr in (left, right):
        pl.semaphore_signal(
            barrier_sem, inc=1,
            device_id=(nbr,), device_id_type=pl.DeviceIdType.MESH,
        )
    pl.semaphore_wait(barrier_sem, 2)

    q = jnp.dot(x_ref[:, :], wq_ref[:, :],
                preferred_element_type=jnp.float32)
    q = q.astype(jnp.bfloat16)

    qi = lax.broadcasted_iota(jnp.int32, (SQ, SKV), 0)
    ki = lax.broadcasted_iota(jnp.int32, (SQ, SKV), 1)
    mask = (jnp.abs(qi - ki) <= 128) | (ki < 32) | (qi < 32)

    for h in range(HQ_LOCAL):
        qh = q[:, h * DH:(h + 1) * DH]
        kh = k_ref[h]
        s = lax.dot_general(
            qh, kh,
            dimension_numbers=(((1,), (1,)), ((), ())),
            preferred_element_type=jnp.float32,
        ) * SCALE
        s = jnp.where(mask, s, -1e9)
        m = jnp.max(s, axis=-1, keepdims=True)
        w = jnp.exp(s - m)
        w = w / jnp.sum(w, axis=-1, keepdims=True)
        ch = jnp.dot(w.astype(jnp.bfloat16), v_ref[h],
                     preferred_element_type=jnp.float32)
        ctx_ref[:, h * DH:(h + 1) * DH] = ch.astype(jnp.bfloat16)

    pb_ref[:, :] = jnp.dot(ctx_ref[:, :], wo_ref[:, :],
                           preferred_element_type=jnp.float32)

    def pchunk(c):
        return pb_ref[pl.ds(c * CHUNK, CHUNK), :]

    fwd_ref[0, :, :] = pchunk(_mod4(my - 1)).astype(jnp.bfloat16)
    for s in range(N_DEV - 1):
        rdma = pltpu.make_async_remote_copy(
            src_ref=fwd_ref.at[s],
            dst_ref=rs_recv_ref.at[s],
            send_sem=rs_send_sems.at[s],
            recv_sem=rs_recv_sems.at[s],
            device_id=(right,),
            device_id_type=pl.DeviceIdType.MESH,
        )
        rdma.start()
        rdma.wait()
        acc = rs_recv_ref[s].astype(jnp.float32) + pchunk(_mod4(my - 2 - s))
        if s < N_DEV - 2:
            fwd_ref[s + 1, :, :] = acc.astype(jnp.bfloat16)
        else:
            out_ref[pl.ds(my * CHUNK, CHUNK), :] = acc
            mychunk_ref[:, :] = acc.astype(jnp.bfloat16)

    for s in range(N_DEV - 1):
        src = mychunk_ref if s == 0 else ag_recv_ref.at[s - 1]
        rdma = pltpu.make_async_remote_copy(
            src_ref=src,
            dst_ref=ag_recv_ref.at[s],
            send_sem=ag_send_sems.at[s],
            recv_sem=ag_recv_sems.at[s],
            device_id=(right,),
            device_id_type=pl.DeviceIdType.MESH,
        )
        rdma.start()
        rdma.wait()
        c = _mod4(my - 1 - s)
        out_ref[pl.ds(c * CHUNK, CHUNK), :] = ag_recv_ref[s].astype(jnp.float32)


def kernel(x, Wq, K_ext, V_ext, Wo):
    my = lax.axis_index("i")

    xb = x[0].astype(jnp.bfloat16)
    Wqb = Wq.astype(jnp.bfloat16)
    Wob = Wo.astype(jnp.bfloat16)
    Kh = lax.dynamic_slice_in_dim(K_ext[0], my * HQ_LOCAL, HQ_LOCAL, axis=1)
    Vh = lax.dynamic_slice_in_dim(V_ext[0], my * HQ_LOCAL, HQ_LOCAL, axis=1)
    Kh = jnp.transpose(Kh, (1, 0, 2)).astype(jnp.bfloat16)
    Vh = jnp.transpose(Vh, (1, 0, 2)).astype(jnp.bfloat16)

    out = pl.pallas_call(
        _body,
        out_shape=jax.ShapeDtypeStruct((SQ, D_MODEL), jnp.float32),
        in_specs=[pl.BlockSpec(memory_space=pltpu.VMEM)] * 5,
        out_specs=pl.BlockSpec(memory_space=pltpu.VMEM),
        scratch_shapes=[
            pltpu.VMEM((SQ, HQ_LOCAL * DH), jnp.bfloat16),
            pltpu.VMEM((SQ, D_MODEL), jnp.float32),
            pltpu.VMEM((N_DEV - 1, CHUNK, D_MODEL), jnp.bfloat16),
            pltpu.VMEM((N_DEV - 1, CHUNK, D_MODEL), jnp.bfloat16),
            pltpu.VMEM((CHUNK, D_MODEL), jnp.bfloat16),
            pltpu.VMEM((N_DEV - 1, CHUNK, D_MODEL), jnp.bfloat16),
            pltpu.SemaphoreType.DMA((N_DEV - 1,)),
            pltpu.SemaphoreType.DMA((N_DEV - 1,)),
            pltpu.SemaphoreType.DMA((N_DEV - 1,)),
            pltpu.SemaphoreType.DMA((N_DEV - 1,)),
        ],
        compiler_params=pltpu.CompilerParams(collective_id=0),
    )(xb, Wqb, Kh, Vh, Wob)
    return out[None]


# device time: 60315 ns/iter; 2.9462x vs baseline; 1.3274x over previous
import jax
import jax.numpy as jnp
from jax import lax
from jax.experimental import pallas as pl
from jax.experimental.pallas import tpu as pltpu

N_DEV = 4
SQ = 1024
SKV = 1024
HQ_LOCAL = 8
DH = 128
D_MODEL = 1024
CHUNK = SQ // N_DEV
SCALE = 0.08838834764831843
MESH = pl.DeviceIdType.MESH


def _mod4(v):
    return lax.rem(v + 2 * N_DEV, N_DEV)


def _body(x_ref, wq_ref, k_ref, v_ref, wo_ref, out_ref,
          sendbuf_ref, srecv_ref, mychunk_ref, brecv_ref, pown_ref,
          ssend_sems, srecv_sems, bsend_sems, brecv_sems):
    my = lax.axis_index("i")

    barrier_sem = pltpu.get_barrier_semaphore()
    for j in range(1, N_DEV):
        pl.semaphore_signal(
            barrier_sem, inc=1,
            device_id=(_mod4(my + j),), device_id_type=MESH,
        )
    pl.semaphore_wait(barrier_sem, N_DEV - 1)

    ki = lax.broadcasted_iota(jnp.int32, (CHUNK, SKV), 1)
    qi_rel = lax.broadcasted_iota(jnp.int32, (CHUNK, SKV), 0)

    def partial_chunk(c):
        rows = pl.ds(c * CHUNK, CHUNK)
        q = jnp.dot(x_ref[rows, :], wq_ref[:, :],
                    preferred_element_type=jnp.float32).astype(jnp.bfloat16)
        qi = qi_rel + c * CHUNK
        mask = (jnp.abs(qi - ki) <= 128) | (ki < 32) | (qi < 32)
        ctx = []
        for h in range(HQ_LOCAL):
            qh = q[:, h * DH:(h + 1) * DH]
            s = lax.dot_general(
                qh, k_ref[h],
                dimension_numbers=(((1,), (1,)), ((), ())),
                preferred_element_type=jnp.float32,
            ) * SCALE
            s = jnp.where(mask, s, -1e9)
            m = jnp.max(s, axis=-1, keepdims=True)
            w = jnp.exp(s - m)
            w = w / jnp.sum(w, axis=-1, keepdims=True)
            ch = jnp.dot(w.astype(jnp.bfloat16), v_ref[h],
                         preferred_element_type=jnp.float32)
            ctx.append(ch.astype(jnp.bfloat16))
        ctx = jnp.concatenate(ctx, axis=1)
        return jnp.dot(ctx, wo_ref[:, :], preferred_element_type=jnp.float32)

    scatter = []
    for j in range(N_DEV - 1):
        tgt = _mod4(my + 1 + j)
        sendbuf_ref[j, :, :] = partial_chunk(tgt).astype(jnp.bfloat16)
        rdma = pltpu.make_async_remote_copy(
            src_ref=sendbuf_ref.at[j],
            dst_ref=srecv_ref.at[2 - j],
            send_sem=ssend_sems.at[j],
            recv_sem=srecv_sems.at[2 - j],
            device_id=(tgt,), device_id_type=MESH,
        )
        rdma.start()
        scatter.append(rdma)

    pown_ref[:, :] = partial_chunk(my)

    acc = pown_ref[:, :]
    for i in range(N_DEV - 1):
        recv = pltpu.make_async_remote_copy(
            src_ref=sendbuf_ref.at[0],
            dst_ref=srecv_ref.at[i],
            send_sem=ssend_sems.at[0],
            recv_sem=srecv_sems.at[i],
            device_id=(my,), device_id_type=MESH,
        )
        recv.wait_recv()
        acc = acc + srecv_ref[i].astype(jnp.float32)
    out_ref[pl.ds(my * CHUNK, CHUNK), :] = acc
    mychunk_ref[:, :] = acc.astype(jnp.bfloat16)

    bcasts = []
    for j in range(N_DEV - 1):
        tgt = _mod4(my + 1 + j)
        rdma = pltpu.make_async_remote_copy(
            src_ref=mychunk_ref,
            dst_ref=brecv_ref.at[2 - j],
            send_sem=bsend_sems.at[j],
            recv_sem=brecv_sems.at[2 - j],
            device_id=(tgt,), device_id_type=MESH,
        )
        rdma.start()
        bcasts.append(rdma)

    for i in range(N_DEV - 1):
        recv = pltpu.make_async_remote_copy(
            src_ref=mychunk_ref,
            dst_ref=brecv_ref.at[i],
            send_sem=bsend_sems.at[0],
            recv_sem=brecv_sems.at[i],
            device_id=(my,), device_id_type=MESH,
        )
        recv.wait_recv()
        src_chip = _mod4(my + 1 + i)
        out_ref[pl.ds(src_chip * CHUNK, CHUNK), :] = (
            brecv_ref[i].astype(jnp.float32))

    for rdma in scatter + bcasts:
        rdma.wait_send()


def kernel(x, Wq, K_ext, V_ext, Wo):
    my = lax.axis_index("i")

    xb = x[0].astype(jnp.bfloat16)
    Wqb = Wq.astype(jnp.bfloat16)
    Wob = Wo.astype(jnp.bfloat16)
    Kh = lax.dynamic_slice_in_dim(K_ext[0], my * HQ_LOCAL, HQ_LOCAL, axis=1)
    Vh = lax.dynamic_slice_in_dim(V_ext[0], my * HQ_LOCAL, HQ_LOCAL, axis=1)
    Kh = jnp.transpose(Kh, (1, 0, 2)).astype(jnp.bfloat16)
    Vh = jnp.transpose(Vh, (1, 0, 2)).astype(jnp.bfloat16)

    out = pl.pallas_call(
        _body,
        out_shape=jax.ShapeDtypeStruct((SQ, D_MODEL), jnp.float32),
        in_specs=[pl.BlockSpec(memory_space=pltpu.VMEM)] * 5,
        out_specs=pl.BlockSpec(memory_space=pltpu.VMEM),
        scratch_shapes=[
            pltpu.VMEM((N_DEV - 1, CHUNK, D_MODEL), jnp.bfloat16),
            pltpu.VMEM((N_DEV - 1, CHUNK, D_MODEL), jnp.bfloat16),
            pltpu.VMEM((CHUNK, D_MODEL), jnp.bfloat16),
            pltpu.VMEM((N_DEV - 1, CHUNK, D_MODEL), jnp.bfloat16),
            pltpu.VMEM((CHUNK, D_MODEL), jnp.float32),
            pltpu.SemaphoreType.DMA((N_DEV - 1,)),
            pltpu.SemaphoreType.DMA((N_DEV - 1,)),
            pltpu.SemaphoreType.DMA((N_DEV - 1,)),
            pltpu.SemaphoreType.DMA((N_DEV - 1,)),
        ],
        compiler_params=pltpu.CompilerParams(collective_id=0),
    )(xb, Wqb, Kh, Vh, Wob)
    return out[None]
